# Initial kernel scaffold; baseline (speedup 1.0000x reference)
import jax
import jax.numpy as jnp
from jax.experimental import pallas as pl


def kernel(x, adj_t, mu_W1, mu_b1, mu_W2, mu_b2, sig_W1, sig_b1, sig_W2, sig_b2, pi_W1, pi_b1, pi_W2, pi_b2, norm_noise, u_pi, u_pp):
    raise NotImplementedError("write your pallas kernel here")


# speedup vs baseline: 1.0750x; 1.0750x over previous
import jax
import jax.numpy as jnp
import numpy as np
from jax.experimental import pallas as pl
from jax.experimental.pallas import tpu as pltpu

_B = 8
_N = 1024
_IN = 256
_HID = 128
_K = 10
_TAU = 0.5
_ROWS = 256
_CLIP = float(-np.log(np.float32(1e-8)))


def _phase_a_body(adj_ref, x_ref, w1_ref, b1_ref, w2_ref, b2_ref,
                  upi_ref, noise_ref, s_ref):
    adj = adj_ref[...]
    xb = x_ref[0]
    ax = jnp.dot(adj, xb, preferred_element_type=jnp.float32)
    h = jnp.maximum(jnp.dot(ax, w1_ref[...],
                            preferred_element_type=jnp.float32)
                    + b1_ref[...], 0.0)
    ah = jnp.dot(adj, h, preferred_element_type=jnp.float32)
    out = jnp.dot(ah, w2_ref[...],
                  preferred_element_type=jnp.float32) + b2_ref[...]
    g = -jnp.log(-jnp.log(upi_ref[0]))
    lane = jax.lax.broadcasted_iota(jnp.int32, (_N, 128), 1)
    z = jnp.where(lane < _K, (out + g) / _TAU, -1e30)
    z = z - jnp.max(z, axis=1, keepdims=True)
    e = jnp.exp(z)
    y = e / jnp.sum(e, axis=1, keepdims=True)
    w = jnp.roll(y, _K, axis=1) + noise_ref[0] * jnp.roll(y, 2 * _K, axis=1)
    s_ref[0] = jnp.sum(out * w, axis=1, keepdims=True)


def _phase_b_body(u_ref, srow_ref, sdup_ref, e_ref, o_ref):
    u = u_ref[0]
    t = jnp.log(u)
    tr = jnp.roll(t, -1, axis=1)
    q = t / tr
    sim = srow_ref[0] * sdup_ref[0]
    ell = jnp.clip(sim, -_CLIP, _CLIP)
    r2 = jnp.exp(-2.0 * ell) * (q * q)
    val = 1.0 / (1.0 + r2)
    o_ref[0] = jnp.dot(val.astype(jnp.bfloat16), e_ref[...],
                       preferred_element_type=jnp.float32)


def kernel(x, adj_t, mu_W1, mu_b1, mu_W2, mu_b2, sig_W1, sig_b1, sig_W2,
           sig_b2, pi_W1, pi_b1, pi_W2, pi_b2, norm_noise, u_pi, u_pp):
    f32 = jnp.float32
    w1 = jnp.concatenate([pi_W1, mu_W1, sig_W1], axis=1)
    b1 = jnp.concatenate([pi_b1, mu_b1, sig_b1]).reshape(1, 3 * _HID)
    w2 = jnp.zeros((3 * _HID, 128), f32)
    w2 = w2.at[0:_HID, 0:_K].set(pi_W2)
    w2 = w2.at[_HID:2 * _HID, _K:2 * _K].set(mu_W2)
    w2 = w2.at[2 * _HID:3 * _HID, 2 * _K:3 * _K].set(sig_W2)
    b2 = jnp.zeros((1, 128), f32)
    b2 = b2.at[0, 0:_K].set(pi_b2)
    b2 = b2.at[0, _K:2 * _K].set(mu_b2)
    b2 = b2.at[0, 2 * _K:3 * _K].set(sig_b2)
    upi_pad = jnp.concatenate(
        [u_pi, jnp.full((_B, _N, 128 - _K), 0.5, f32)], axis=-1)
    noise = norm_noise.reshape(_B, _N, 1)

    s = pl.pallas_call(
        _phase_a_body,
        grid=(_B,),
        in_specs=[
            pl.BlockSpec((_N, _N), lambda b: (0, 0)),
            pl.BlockSpec((1, _N, _IN), lambda b: (b, 0, 0)),
            pl.BlockSpec((_IN, 3 * _HID), lambda b: (0, 0)),
            pl.BlockSpec((1, 3 * _HID), lambda b: (0, 0)),
            pl.BlockSpec((3 * _HID, 128), lambda b: (0, 0)),
            pl.BlockSpec((1, 128), lambda b: (0, 0)),
            pl.BlockSpec((1, _N, 128), lambda b: (b, 0, 0)),
            pl.BlockSpec((1, _N, 1), lambda b: (b, 0, 0)),
        ],
        out_specs=pl.BlockSpec((1, _N, 1), lambda b: (b, 0, 0)),
        out_shape=jax.ShapeDtypeStruct((_B, _N, 1), f32),
    )(adj_t, x, w1, b1, w2, b2, upi_pad, noise)

    sdup = jnp.repeat(s[..., 0], 2, axis=-1).reshape(_B, 1, 2 * _N)
    u2 = u_pp.reshape(_B, _N, 2 * _N)
    esel = (jax.lax.broadcasted_iota(jnp.int32, (2 * _N, _N), 0)
            == 2 * jax.lax.broadcasted_iota(jnp.int32, (2 * _N, _N), 1)
            ).astype(jnp.bfloat16)

    a = pl.pallas_call(
        _phase_b_body,
        grid=(_B, _N // _ROWS),
        in_specs=[
            pl.BlockSpec((1, _ROWS, 2 * _N), lambda b, i: (b, i, 0)),
            pl.BlockSpec((1, _ROWS, 1), lambda b, i: (b, i, 0)),
            pl.BlockSpec((1, 1, 2 * _N), lambda b, i: (b, 0, 0)),
            pl.BlockSpec((2 * _N, _N), lambda b, i: (0, 0)),
        ],
        out_specs=pl.BlockSpec((1, _ROWS, _N), lambda b, i: (b, i, 0)),
        out_shape=jax.ShapeDtypeStruct((_B, _N, _N), f32),
    )(u2, s, sdup, esel)
    return a


# speedup vs baseline: 2.0444x; 1.9018x over previous
import jax
import jax.numpy as jnp
import numpy as np
from jax.experimental import pallas as pl
from jax.experimental.pallas import tpu as pltpu

_B = 8
_N = 1024
_IN = 256
_HID = 128
_K = 10
_TAU = 0.5
_ROWS = 256
_CLIP = float(-np.log(np.float32(1e-8)))


def _phase_a_body(adj_ref, x_ref, w1_ref, b1_ref, w2_ref, b2_ref,
                  upi_ref, noise_ref, s_ref):
    adj = adj_ref[...]
    xb = x_ref[0]
    ax = jnp.dot(adj, xb, preferred_element_type=jnp.float32)
    h = jnp.maximum(jnp.dot(ax, w1_ref[...],
                            preferred_element_type=jnp.float32)
                    + b1_ref[...], 0.0)
    ah = jnp.dot(adj, h, preferred_element_type=jnp.float32)
    out = jnp.dot(ah, w2_ref[...],
                  preferred_element_type=jnp.float32) + b2_ref[...]
    g = -jnp.log(-jnp.log(upi_ref[0]))
    lane = jax.lax.broadcasted_iota(jnp.int32, (_N, 128), 1)
    z = jnp.where(lane < _K, (out + g) / _TAU, -1e30)
    z = z - jnp.max(z, axis=1, keepdims=True)
    e = jnp.exp(z)
    y = e / jnp.sum(e, axis=1, keepdims=True)
    w = jnp.roll(y, _K, axis=1) + noise_ref[0] * jnp.roll(y, 2 * _K, axis=1)
    s_ref[0] = jnp.sum(out * w, axis=1, keepdims=True)


def _phase_b_body(u_ref, srow_ref, scol_ref, o_ref):
    u0 = u_ref[0, :, 0, :]
    u1 = u_ref[0, :, 1, :]
    q = jnp.log(u0) / jnp.log(u1)
    sim = srow_ref[0] * scol_ref[0]
    ell = jnp.clip(sim, -_CLIP, _CLIP)
    r2 = jnp.exp(-2.0 * ell) * (q * q)
    o_ref[0] = 1.0 / (1.0 + r2)


def kernel(x, adj_t, mu_W1, mu_b1, mu_W2, mu_b2, sig_W1, sig_b1, sig_W2,
           sig_b2, pi_W1, pi_b1, pi_W2, pi_b2, norm_noise, u_pi, u_pp):
    f32 = jnp.float32
    w1 = jnp.concatenate([pi_W1, mu_W1, sig_W1], axis=1)
    b1 = jnp.concatenate([pi_b1, mu_b1, sig_b1]).reshape(1, 3 * _HID)
    w2 = jnp.zeros((3 * _HID, 128), f32)
    w2 = w2.at[0:_HID, 0:_K].set(pi_W2)
    w2 = w2.at[_HID:2 * _HID, _K:2 * _K].set(mu_W2)
    w2 = w2.at[2 * _HID:3 * _HID, 2 * _K:3 * _K].set(sig_W2)
    b2 = jnp.zeros((1, 128), f32)
    b2 = b2.at[0, 0:_K].set(pi_b2)
    b2 = b2.at[0, _K:2 * _K].set(mu_b2)
    b2 = b2.at[0, 2 * _K:3 * _K].set(sig_b2)
    upi_pad = jnp.concatenate(
        [u_pi, jnp.full((_B, _N, 128 - _K), 0.5, f32)], axis=-1)
    noise = norm_noise.reshape(_B, _N, 1)

    s = pl.pallas_call(
        _phase_a_body,
        grid=(_B,),
        in_specs=[
            pl.BlockSpec((_N, _N), lambda b: (0, 0)),
            pl.BlockSpec((1, _N, _IN), lambda b: (b, 0, 0)),
            pl.BlockSpec((_IN, 3 * _HID), lambda b: (0, 0)),
            pl.BlockSpec((1, 3 * _HID), lambda b: (0, 0)),
            pl.BlockSpec((3 * _HID, 128), lambda b: (0, 0)),
            pl.BlockSpec((1, 128), lambda b: (0, 0)),
            pl.BlockSpec((1, _N, 128), lambda b: (b, 0, 0)),
            pl.BlockSpec((1, _N, 1), lambda b: (b, 0, 0)),
        ],
        out_specs=pl.BlockSpec((1, _N, 1), lambda b: (b, 0, 0)),
        out_shape=jax.ShapeDtypeStruct((_B, _N, 1), f32),
    )(adj_t, x, w1, b1, w2, b2, upi_pad, noise)

    scol = s.reshape(_B, 1, _N)
    upt = jnp.transpose(u_pp, (0, 1, 3, 2))

    a = pl.pallas_call(
        _phase_b_body,
        grid=(_B, _N // _ROWS),
        in_specs=[
            pl.BlockSpec((1, _ROWS, 2, _N), lambda b, i: (b, i, 0, 0)),
            pl.BlockSpec((1, _ROWS, 1), lambda b, i: (b, i, 0)),
            pl.BlockSpec((1, 1, _N), lambda b, i: (b, 0, 0)),
        ],
        out_specs=pl.BlockSpec((1, _ROWS, _N), lambda b, i: (b, i, 0)),
        out_shape=jax.ShapeDtypeStruct((_B, _N, _N), f32),
    )(upt, s, scol)
    return a


# speedup vs baseline: 2.7680x; 1.3539x over previous
import jax
import jax.numpy as jnp
import numpy as np
from jax import lax
from jax.experimental import pallas as pl
from jax.experimental.pallas import tpu as pltpu

_B = 8
_N = 1024
_IN = 256
_HID = 128
_K = 10
_TAU = 0.5
_ROWS = 512
_NT = _N // _ROWS
_NTILES = _B * _NT
_NSLOT = 3
_CLIP = float(-np.log(np.float32(1e-8)))
_CLIP2 = float(2.0 * np.log2(np.e) * _CLIP)
_NEG2LOG2E = float(-2.0 * np.log2(np.e))


def _body(adj_ref, x_ref, w1pi_ref, w1mu_ref, w1sig_ref, b1_ref,
          w2pi_ref, w2mu_ref, w2sig_ref, b2_ref, upi_ref, noise_ref,
          u_hbm, o_ref, u0b, u1b, srow_sc, scol_sc, sems):
    f32 = jnp.float32
    bf = jnp.bfloat16
    b = pl.program_id(0)
    i = pl.program_id(1)

    def _start(slot, bb, tt):
        pltpu.make_async_copy(
            u_hbm.at[bb, pl.ds(tt * _ROWS, _ROWS), 0, :],
            u0b.at[slot], sems.at[slot, 0]).start()
        pltpu.make_async_copy(
            u_hbm.at[bb, pl.ds(tt * _ROWS, _ROWS), 1, :],
            u1b.at[slot], sems.at[slot, 1]).start()

    @pl.when(i == 0)
    def _phase_a():
        @pl.when(b == 0)
        def _prime():
            _start(0, 0, 0)
            if _NT > 1:
                _start(1, 0, 1)

        adjb = adj_ref[...].astype(bf)
        xb = x_ref[0].astype(bf)
        ax = jnp.dot(adjb, xb, preferred_element_type=f32).astype(bf)

        def head(w1_r, brow, w2_r, b2row):
            h = jnp.maximum(
                jnp.dot(ax, w1_r[...], preferred_element_type=f32)
                + b1_ref[brow:brow + 1, :], 0.0)
            ah = jnp.dot(adjb, h.astype(bf), preferred_element_type=f32)
            return (jnp.dot(ah.astype(bf), w2_r[...],
                            preferred_element_type=f32)
                    + b2_ref[b2row:b2row + 1, :])

        o_pi = head(w1pi_ref, 0, w2pi_ref, 0)
        o_mu = head(w1mu_ref, 1, w2mu_ref, 1)
        o_sig = head(w1sig_ref, 2, w2sig_ref, 2)

        slab = jnp.concatenate(
            [upi_ref[pl.ds(b + _B * k, 1), :] for k in range(_K)], axis=0)
        g = jnp.transpose(-jnp.log(-jnp.log(slab)))
        z = (o_pi + g) * f32(1.0 / _TAU)
        z = z - jnp.max(z, axis=1, keepdims=True)
        e = jnp.exp(z)
        y = e / jnp.sum(e, axis=1, keepdims=True)
        s_val = (jnp.sum(o_mu * y, axis=1, keepdims=True)
                 + noise_ref[0] * jnp.sum(o_sig * y, axis=1, keepdims=True))
        srow_sc[...] = s_val * f32(_NEG2LOG2E)
        scol_sc[...] = jnp.transpose(s_val)

    @pl.when(i > 0)
    def _phase_b():
        t = b * _NT + (i - 1)
        slot = lax.rem(t, _NSLOT)

        @pl.when(t + 2 < _NTILES)
        def _prefetch():
            t2 = t + 2
            _start(lax.rem(t2, _NSLOT), lax.div(t2, _NT), lax.rem(t2, _NT))

        pltpu.make_async_copy(
            u_hbm.at[b, pl.ds((i - 1) * _ROWS, _ROWS), 0, :],
            u0b.at[slot], sems.at[slot, 0]).wait()
        pltpu.make_async_copy(
            u_hbm.at[b, pl.ds((i - 1) * _ROWS, _ROWS), 1, :],
            u1b.at[slot], sems.at[slot, 1]).wait()

        l0 = jnp.log2(u0b[slot])
        l1 = jnp.log2(u1b[slot])
        srow = srow_sc[pl.ds((i - 1) * _ROWS, _ROWS), :]
        e2 = lax.exp2(jnp.clip(srow * scol_sc[...], -_CLIP2, _CLIP2))
        n0 = l0 * l0 * e2
        n1 = l1 * l1
        o_ref[0] = n1 / (n1 + n0)


def kernel(x, adj_t, mu_W1, mu_b1, mu_W2, mu_b2, sig_W1, sig_b1, sig_W2,
           sig_b2, pi_W1, pi_b1, pi_W2, pi_b2, norm_noise, u_pi, u_pp):
    f32 = jnp.float32
    bf = jnp.bfloat16
    b1 = jnp.stack([pi_b1, mu_b1, sig_b1])
    b2 = jnp.stack([pi_b2, mu_b2, sig_b2])
    w1pi, w1mu, w1sig = (w.astype(bf) for w in (pi_W1, mu_W1, sig_W1))
    w2pi, w2mu, w2sig = (w.astype(bf) for w in (pi_W2, mu_W2, sig_W2))
    upi = jnp.transpose(u_pi, (2, 0, 1)).reshape(_K * _B, _N)
    noise = norm_noise.reshape(_B, _N, 1)
    upt = jnp.transpose(u_pp, (0, 1, 3, 2))

    const = lambda *idx: (lambda b, i: idx)
    a = pl.pallas_call(
        _body,
        grid=(_B, 1 + _NT),
        in_specs=[
            pl.BlockSpec((_N, _N), const(0, 0)),
            pl.BlockSpec((1, _N, _IN), lambda b, i: (b, 0, 0)),
            pl.BlockSpec((_IN, _HID), const(0, 0)),
            pl.BlockSpec((_IN, _HID), const(0, 0)),
            pl.BlockSpec((_IN, _HID), const(0, 0)),
            pl.BlockSpec((3, _HID), const(0, 0)),
            pl.BlockSpec((_HID, _K), const(0, 0)),
            pl.BlockSpec((_HID, _K), const(0, 0)),
            pl.BlockSpec((_HID, _K), const(0, 0)),
            pl.BlockSpec((3, _K), const(0, 0)),
            pl.BlockSpec((_K * _B, _N), const(0, 0)),
            pl.BlockSpec((1, _N, 1), lambda b, i: (b, 0, 0)),
            pl.BlockSpec(memory_space=pltpu.MemorySpace.HBM),
        ],
        out_specs=pl.BlockSpec(
            (1, _ROWS, _N), lambda b, i: (b, jnp.maximum(i - 1, 0), 0)),
        out_shape=jax.ShapeDtypeStruct((_B, _N, _N), f32),
        scratch_shapes=[
            pltpu.VMEM((_NSLOT, _ROWS, _N), f32),
            pltpu.VMEM((_NSLOT, _ROWS, _N), f32),
            pltpu.VMEM((_N, 1), f32),
            pltpu.VMEM((1, _N), f32),
            pltpu.SemaphoreType.DMA((_NSLOT, 2)),
        ],
    )(adj_t, x, w1pi, w1mu, w1sig, b1, w2pi, w2mu, w2sig, b2, upi, noise,
      upt)
    return a
